# baseline (speedup 1.0000x reference)
import functools

import jax
import jax.numpy as jnp
from jax import lax
from jax.experimental import pallas as pl
from jax.experimental.pallas import tpu as pltpu
from jax.experimental.pallas import tpu_sc as plsc

B = 1024
L = 200
E = 128
Y = 50

NC = 2
NS = 16
NW = NC * NS
BPW = B // NW
NLANE = 16
EV = E // NLANE

_mesh = plsc.VectorSubcoreMesh(core_axis_name="c", subcore_axis_name="s")


@functools.partial(
    pl.kernel,
    mesh=_mesh,
    out_type=jax.ShapeDtypeStruct((B, E), jnp.float32),
    scratch_types=[
        pltpu.VMEM((BPW * L,), jnp.int32),
        pltpu.VMEM((L, E), jnp.float32),
        pltpu.VMEM((L, E), jnp.float32),
        pltpu.VMEM((BPW, E), jnp.float32),
        pltpu.SemaphoreType.DMA,
        pltpu.SemaphoreType.DMA,
    ],
)
def _pool_sc(x_hbm, w_hbm, out_hbm, idx_v, buf0, buf1, pooled_v, sem0, sem1):
    wid = lax.axis_index("s") * NC + lax.axis_index("c")
    base = wid * BPW

    pltpu.sync_copy(x_hbm.at[pl.ds(base * L, BPW * L)], idx_v)

    def issue(r, buf, sem):
        pltpu.async_copy(
            w_hbm.at[idx_v.at[pl.ds(r * L, 128)]], buf.at[pl.ds(0, 128)], sem)
        pltpu.async_copy(
            w_hbm.at[idx_v.at[pl.ds(r * L + 128, L - 128)]],
            buf.at[pl.ds(128, L - 128)], sem)

    def consume(r, buf, sem):
        pltpu.make_async_copy(w_hbm.at[pl.ds(0, L)], buf, sem).wait()

        def acc_body(j, accs):
            out = []
            for e, a in enumerate(accs):
                sl = pl.ds(e * NLANE, NLANE)
                s01 = buf[4 * j, sl] + buf[4 * j + 1, sl]
                s23 = buf[4 * j + 2, sl] + buf[4 * j + 3, sl]
                out.append(a + (s01 + s23))
            return tuple(out)

        accs = lax.fori_loop(
            0, L // 4, acc_body,
            tuple(jnp.zeros((NLANE,), jnp.float32) for _ in range(EV)))
        for e in range(EV):
            pooled_v[r, pl.ds(e * NLANE, NLANE)] = accs[e]

    issue(0, buf0, sem0)

    def pair_body(r2, carry):
        r = 2 * r2
        issue(r + 1, buf1, sem1)
        consume(r, buf0, sem0)
        issue(r + 2, buf0, sem0)
        consume(r + 1, buf1, sem1)
        return carry

    lax.fori_loop(0, BPW // 2 - 1, pair_body, 0)
    issue(BPW - 1, buf1, sem1)
    consume(BPW - 2, buf0, sem0)
    consume(BPW - 1, buf1, sem1)

    pltpu.sync_copy(pooled_v, out_hbm.at[pl.ds(base, BPW)])


def _dense_tc(p_ref, w_ref, b_ref, o_ref):
    o_ref[...] = lax.dot_general(
        p_ref[...], w_ref[...], (((1,), (1,)), ((), ())),
        preferred_element_type=jnp.float32) + b_ref[...]


def kernel(x, W, fc_w, fc_b):
    xf = x.reshape(B * L).astype(jnp.int32)
    pooled = _pool_sc(xf, W)
    out = pl.pallas_call(
        _dense_tc,
        out_shape=jax.ShapeDtypeStruct((B, Y), jnp.float32),
    )(pooled, fc_w, fc_b.reshape(1, Y))
    return out


# speedup vs baseline: 1.1283x; 1.1283x over previous
import functools

import jax
import jax.numpy as jnp
from jax import lax
from jax.experimental import pallas as pl
from jax.experimental.pallas import tpu as pltpu
from jax.experimental.pallas import tpu_sc as plsc

B = 1024
L = 200
E = 128
Y = 50

NC = 2
NS = 16
NW = NC * NS
BPW = B // NW
NLANE = 16
EV = E // NLANE
NBUF = 4

_mesh = plsc.VectorSubcoreMesh(core_axis_name="c", subcore_axis_name="s")


@functools.partial(
    pl.kernel,
    mesh=_mesh,
    out_type=jax.ShapeDtypeStruct((B, E), jnp.float32),
    scratch_types=[
        pltpu.VMEM((BPW * L,), jnp.int32),
        pltpu.VMEM((NBUF, L, E), jnp.float32),
        pltpu.VMEM((BPW, E), jnp.float32),
    ] + [pltpu.SemaphoreType.DMA] * NBUF,
)
def _pool_sc(x_hbm, w_hbm, out_hbm, idx_v, bufs, pooled_v, *sems):
    wid = lax.axis_index("s") * NC + lax.axis_index("c")
    base = wid * BPW

    pltpu.sync_copy(x_hbm.at[pl.ds(base * L, BPW * L)], idx_v)

    def issue(r, b, sem):
        pltpu.async_copy(
            w_hbm.at[idx_v.at[pl.ds(r * L, 128)]],
            bufs.at[b, pl.ds(0, 128)], sem)
        pltpu.async_copy(
            w_hbm.at[idx_v.at[pl.ds(r * L + 128, L - 128)]],
            bufs.at[b, pl.ds(128, L - 128)], sem)

    def consume(r, b, sem):
        pltpu.make_async_copy(w_hbm.at[pl.ds(0, L)], bufs.at[b], sem).wait()

        def acc_body(j, accs):
            out = []
            for e, a in enumerate(accs):
                sl = pl.ds(e * NLANE, NLANE)
                s01 = bufs[b, 4 * j, sl] + bufs[b, 4 * j + 1, sl]
                s23 = bufs[b, 4 * j + 2, sl] + bufs[b, 4 * j + 3, sl]
                out.append(a + (s01 + s23))
            return tuple(out)

        accs = lax.fori_loop(
            0, L // 4, acc_body,
            tuple(jnp.zeros((NLANE,), jnp.float32) for _ in range(EV)))
        for e in range(EV):
            pooled_v[r, pl.ds(e * NLANE, NLANE)] = accs[e]

    for b in range(NBUF - 1):
        issue(b, b, sems[b])

    def grp_body(g, carry):
        for b in range(NBUF):
            r = g * NBUF + b
            nxt = r + NBUF - 1
            nb = (b + NBUF - 1) % NBUF

            @pl.when(nxt < BPW)
            def _():
                issue(nxt, nb, sems[nb])

            consume(r, b, sems[b])
        return carry

    lax.fori_loop(0, BPW // NBUF, grp_body, 0)
    pltpu.sync_copy(pooled_v, out_hbm.at[pl.ds(base, BPW)])


def _dense_tc(p_ref, w_ref, b_ref, o_ref):
    o_ref[...] = lax.dot_general(
        p_ref[...], w_ref[...], (((1,), (1,)), ((), ())),
        preferred_element_type=jnp.float32) + b_ref[...]


def kernel(x, W, fc_w, fc_b):
    xf = x.reshape(B * L).astype(jnp.int32)
    pooled = _pool_sc(xf, W)
    out = pl.pallas_call(
        _dense_tc,
        out_shape=jax.ShapeDtypeStruct((B, Y), jnp.float32),
    )(pooled, fc_w, fc_b.reshape(1, Y))
    return out
